# baseline (speedup 1.0000x reference)
import functools

import jax
import jax.numpy as jnp
from jax import lax
from jax.experimental import pallas as pl
from jax.experimental.pallas import tpu as pltpu
from jax.experimental.pallas import tpu_sc as plsc

N = 32768
R = 128
KWIN = int(N * 0.05)
NB = 256
NV = N // 16
NC = 2
NS = 16
NW = NC * NS
ROWS_PER_W = R // NW
SHIFTS = (24, 16, 8, 0)
MANT = 0x7FFFFFFF


def _body(x_hbm, out_hbm, xbuf, hist):
    wid = lax.axis_index("s") * NC + lax.axis_index("c")
    lane = lax.iota(jnp.int32, 16)
    ones = jnp.ones((16,), jnp.int32)
    zeros_i = jnp.zeros((16,), jnp.int32)

    def hist_slot(bucket):
        return (bucket << 4) + ((bucket + lane) & 15)

    def zero_body(i, c):
        hist[pl.ds(i * 16, 16)] = zeros_i
        return c
    lax.fori_loop(0, (16 * NB) // 16, zero_body, 0)

    def key_of(xv):
        iv = lax.bitcast_convert_type(xv, jnp.int32)
        return iv ^ ((iv >> 31) & MANT)

    for r in range(ROWS_PER_W):
        base = (wid * ROWS_PER_W + r) * N
        pltpu.sync_copy(x_hbm.at[pl.ds(base, N)], xbuf)

        k_rem = jnp.int32(KWIN)
        ncand = jnp.int32(N)
        p = jnp.int32(0)
        U = 16
        for li, s in enumerate(SHIFTS):
            if li == 0:
                @plsc.parallel_loop(0, NV, 1, unroll=U)
                def _(j):
                    key = key_of(xbuf[pl.ds(j * 16, 16)])
                    bucket = (key >> 24) + 128
                    plsc.addupdate_scatter(hist, [hist_slot(bucket)], ones)
            else:
                sp = SHIFTS[li - 1]
                pv = jnp.full((16,), p, jnp.int32)

                @plsc.parallel_loop(0, NV, 1, unroll=U)
                def _(j, s=s, sp=sp, pv=pv):
                    key = key_of(xbuf[pl.ds(j * 16, 16)])
                    m = (key >> sp) == pv
                    bucket = (key >> s) & 255
                    plsc.addupdate_scatter(hist, [hist_slot(bucket)], ones,
                                           mask=m)

            lim = ncand - k_rem
            def scan_body(c, carry):
                cum, bcnt, cle_at, clt_at = carry
                h = zeros_i
                gbase = (c * 16 + lane) << 4
                for w in range(16):
                    h = h + plsc.load_gather(hist, [gbase + ((lane + w) & 15)])
                for l in range(16):
                    hist[pl.ds(c * 256 + l * 16, 16)] = zeros_i
                cle = plsc.cumsum(h) + cum
                clt = cle - h
                cond = clt <= lim
                bcnt = bcnt + plsc.all_reduce_population_count(cond)
                cle_at = jnp.maximum(cle_at, jnp.where(cond, cle, zeros_i))
                clt_at = jnp.maximum(clt_at, jnp.where(cond, clt, zeros_i))
                cum = jnp.max(cle)
                return (cum, bcnt, cle_at, clt_at)

            _, bcnt, cle_at, clt_at = lax.fori_loop(
                0, NB // 16, scan_body,
                (jnp.int32(0), zeros_i, zeros_i, zeros_i))
            bstar = jnp.max(bcnt) - 1
            cle_s = jnp.max(cle_at)
            clt_s = jnp.max(clt_at)
            k_rem = k_rem - (ncand - cle_s)
            ncand = cle_s - clt_s
            if li == 0:
                p = bstar - 128
            else:
                p = (p << 8) | bstar

        tbits = jnp.where(p >= 0, p, p ^ MANT)
        thresh = lax.bitcast_convert_type(jnp.full((16,), tbits, jnp.int32),
                                          jnp.float32)
        zf = jnp.zeros((16,), jnp.float32)

        @plsc.parallel_loop(0, NV, 1, unroll=U)
        def _(j):
            xv = xbuf[pl.ds(j * 16, 16)]
            xbuf[pl.ds(j * 16, 16)] = jnp.where(xv >= thresh, xv, zf)

        pltpu.sync_copy(xbuf, out_hbm.at[pl.ds(base, N)])


_kwta = functools.partial(
    pl.kernel,
    out_type=jax.ShapeDtypeStruct((R * N,), jnp.float32),
    mesh=plsc.VectorSubcoreMesh(core_axis_name="c", subcore_axis_name="s"),
    compiler_params=pltpu.CompilerParams(needs_layout_passes=False),
    scratch_types=[
        pltpu.VMEM((N,), jnp.float32),
        pltpu.VMEM((16 * NB,), jnp.int32),
    ],
)(_body)


def kernel(x):
    return _kwta(x.reshape(-1)).reshape(x.shape)


# speedup vs baseline: 1.0547x; 1.0547x over previous
import functools

import jax
import jax.numpy as jnp
from jax import lax
from jax.experimental import pallas as pl
from jax.experimental.pallas import tpu as pltpu
from jax.experimental.pallas import tpu_sc as plsc

N = 32768
R = 128
KWIN = int(N * 0.05)
NB = 256
NV = N // 16
NC = 2
NS = 16
NW = NC * NS
ROWS_PER_W = R // NW
SHIFTS = (24, 16, 8, 0)
MANT = 0x7FFFFFFF


def _body(x_hbm, out_hbm, xbuf0, xbuf1, hist, histsum,
          sem_in0, sem_in1, sem_out0, sem_out1):
    wid = lax.axis_index("s") * NC + lax.axis_index("c")
    lane = lax.iota(jnp.int32, 16)
    ones = jnp.ones((16,), jnp.int32)
    zeros_i = jnp.zeros((16,), jnp.int32)

    def hist_slot(bucket):
        return (bucket << 4) + ((bucket + lane) & 15)

    def zero_body(i, c):
        hist[pl.ds(i * 16, 16)] = zeros_i
        return c
    lax.fori_loop(0, (16 * NB) // 16, zero_body, 0)

    def key_of(xv):
        iv = lax.bitcast_convert_type(xv, jnp.int32)
        return iv ^ ((iv >> 31) & MANT)

    bufs = (xbuf0, xbuf1)
    sin = (sem_in0, sem_in1)
    sout = (sem_out0, sem_out1)
    bases = [(wid * ROWS_PER_W + r) * N for r in range(ROWS_PER_W)]

    def in_copy(r):
        return pltpu.make_async_copy(
            x_hbm.at[pl.ds(bases[r], N)], bufs[r % 2], sin[r % 2])

    def out_copy(r):
        return pltpu.make_async_copy(
            bufs[r % 2], out_hbm.at[pl.ds(bases[r], N)], sout[r % 2])

    in_copy(0).start()
    for r in range(ROWS_PER_W):
        xbuf = bufs[r % 2]
        in_copy(r).wait()
        if r + 1 < ROWS_PER_W:
            if r >= 1:
                out_copy(r - 1).wait()
            in_copy(r + 1).start()

        k_rem = jnp.int32(KWIN)
        ncand = jnp.int32(N)
        p = jnp.int32(0)
        U = 8
        for li, s in enumerate(SHIFTS):
            if li == 0:
                @plsc.parallel_loop(0, NV, 1, unroll=U)
                def _(j):
                    key = key_of(xbuf[pl.ds(j * 16, 16)])
                    bucket = (key >> 24) + 128
                    plsc.addupdate_scatter(hist, [hist_slot(bucket)], ones)
            else:
                sp = SHIFTS[li - 1]
                pv = jnp.full((16,), p, jnp.int32)

                @plsc.parallel_loop(0, NV, 1, unroll=U)
                def _(j, s=s, sp=sp, pv=pv):
                    key = key_of(xbuf[pl.ds(j * 16, 16)])
                    m = (key >> sp) == pv
                    bucket = (key >> s) & 255
                    plsc.addupdate_scatter(hist, [hist_slot(bucket)], ones,
                                           mask=m)

            @plsc.parallel_loop(0, NB // 16, 1, unroll=2)
            def _(c):
                h = zeros_i
                gbase = (c * 16 + lane) << 4
                for w in range(16):
                    h = h + plsc.load_gather(hist,
                                             [gbase + ((lane + w) & 15)])
                for l in range(16):
                    hist[pl.ds(c * 256 + l * 16, 16)] = zeros_i
                histsum[pl.ds(c * 16, 16)] = h

            lim = ncand - k_rem
            def scan_body(c, carry):
                cum, bcnt, cle_at, clt_at = carry
                h = histsum[pl.ds(c * 16, 16)]
                cle = plsc.cumsum(h) + cum
                clt = cle - h
                cond = clt <= lim
                bcnt = bcnt + plsc.all_reduce_population_count(cond)
                cle_at = jnp.maximum(cle_at, jnp.where(cond, cle, zeros_i))
                clt_at = jnp.maximum(clt_at, jnp.where(cond, clt, zeros_i))
                cum = jnp.max(cle)
                return (cum, bcnt, cle_at, clt_at)

            _, bcnt, cle_at, clt_at = lax.fori_loop(
                0, NB // 16, scan_body,
                (jnp.int32(0), zeros_i, zeros_i, zeros_i))
            bstar = jnp.max(bcnt) - 1
            cle_s = jnp.max(cle_at)
            clt_s = jnp.max(clt_at)
            k_rem = k_rem - (ncand - cle_s)
            ncand = cle_s - clt_s
            if li == 0:
                p = bstar - 128
            else:
                p = (p << 8) | bstar

        tbits = jnp.where(p >= 0, p, p ^ MANT)
        thresh = lax.bitcast_convert_type(jnp.full((16,), tbits, jnp.int32),
                                          jnp.float32)
        zf = jnp.zeros((16,), jnp.float32)

        @plsc.parallel_loop(0, NV, 1, unroll=U)
        def _(j):
            xv = xbuf[pl.ds(j * 16, 16)]
            xbuf[pl.ds(j * 16, 16)] = jnp.where(xv >= thresh, xv, zf)

        out_copy(r).start()

    out_copy(ROWS_PER_W - 2).wait()
    out_copy(ROWS_PER_W - 1).wait()


_kwta = functools.partial(
    pl.kernel,
    out_type=jax.ShapeDtypeStruct((R * N,), jnp.float32),
    mesh=plsc.VectorSubcoreMesh(core_axis_name="c", subcore_axis_name="s"),
    compiler_params=pltpu.CompilerParams(needs_layout_passes=False),
    scratch_types=[
        pltpu.VMEM((N,), jnp.float32),
        pltpu.VMEM((N,), jnp.float32),
        pltpu.VMEM((16 * NB,), jnp.int32),
        pltpu.VMEM((NB,), jnp.int32),
        pltpu.SemaphoreType.DMA,
        pltpu.SemaphoreType.DMA,
        pltpu.SemaphoreType.DMA,
        pltpu.SemaphoreType.DMA,
    ],
)(_body)


def kernel(x):
    return _kwta(x.reshape(-1)).reshape(x.shape)
